# baseline (speedup 1.0000x reference)
import functools

import jax
import jax.numpy as jnp
from jax import lax
from jax.experimental import pallas as pl
from jax.experimental.pallas import tpu as pltpu
from jax.experimental.pallas import tpu_sc as plsc

NC = 2
NS = 16
NW = NC * NS

CHUNK = 80
AUG = 40


def _gather_body(epw, nchunk, x_hbm, src_hbm, xj_hbm, idx_v, rows_v, sem):
    c = lax.axis_index("c")
    s = lax.axis_index("s")
    wid = s * NC + c
    pltpu.sync_copy(src_hbm.at[wid], idx_v)

    def step(j, carry):
        pltpu.async_copy(x_hbm.at[idx_v.at[j]], rows_v, sem).wait()
        pltpu.sync_copy(rows_v, xj_hbm.at[pl.ds(wid * epw + j * CHUNK, CHUNK)])
        return carry

    lax.fori_loop(0, nchunk, step, 0)


def _scatter_body(epw, nchunk, msg_hbm, dst_hbm, zeros_hbm, parts_hbm,
                  acc_sh, idx_v, rows_v):
    c = lax.axis_index("c")
    s = lax.axis_index("s")
    wid = s * NC + c

    @pl.when(s == 0)
    def _():
        pltpu.sync_copy(zeros_hbm, acc_sh)

    plsc.subcore_barrier()
    pltpu.sync_copy(dst_hbm.at[wid], idx_v)

    def step(j, carry):
        pltpu.sync_copy(msg_hbm.at[pl.ds(wid * epw + j * CHUNK, CHUNK)], rows_v)
        pltpu.sync_copy(rows_v, acc_sh.at[idx_v.at[j]], add=True)
        return carry

    lax.fori_loop(0, nchunk, step, 0)
    plsc.subcore_barrier()

    @pl.when(s == 0)
    def _():
        pltpu.sync_copy(acc_sh, parts_hbm.at[c])


def _edge_body(vin, vout, be, e_ref, xj_ref, W1T_ref, b1_ref, W2T_ref,
               b2_ref, out_ref):
    et = e_ref[...].T
    hT = jnp.maximum(
        jnp.dot(W1T_ref[...], et, preferred_element_type=jnp.float32)
        + b1_ref[...], 0.0)
    WeT = jnp.dot(W2T_ref[...], hT.astype(jnp.bfloat16),
                  preferred_element_type=jnp.float32) + b2_ref[...]
    xjT = xj_ref[...].T
    acc = jnp.zeros((vout, be), jnp.float32)
    for i in range(vin):
        acc = acc + WeT[i * vout:(i + 1) * vout, :] * xjT[i:i + 1, :]
    out_ref[...] = jnp.concatenate(
        [acc.T,
         jnp.ones((be, 1), jnp.float32),
         jnp.zeros((be, AUG - vout - 1), jnp.float32)], axis=1)


def _final_body(vout, parts_ref, x_ref, root_ref, bias_ref, out_ref):
    p = parts_ref[...]
    s = p[0, :, 0:vout] + p[1, :, 0:vout]
    cnt = p[0, :, vout:vout + 1] + p[1, :, vout:vout + 1]
    mean = s / jnp.maximum(cnt, 1.0)
    y = mean + jnp.dot(x_ref[...], root_ref[...],
                       preferred_element_type=jnp.float32) + bias_ref[...]
    out_ref[...] = jnp.where(y > 0.0, y, jnp.exp(y) - 1.0)


def kernel(x, e_idx, e, W1, b1, W2, b2, root, bias):
    n, vin = x.shape
    eE, ein = e.shape
    h = W1.shape[1]
    vout = W2.shape[1] // vin

    epw = eE // NW
    nchunk = epw // CHUNK
    assert epw * NW == eE and nchunk * CHUNK == epw

    src3 = e_idx[0].reshape(NW, nchunk, CHUNK)
    dst3 = e_idx[1].reshape(NW, nchunk, CHUNK)

    mesh = plsc.VectorSubcoreMesh(core_axis_name="c", subcore_axis_name="s",
                                  num_cores=NC, num_subcores=NS)

    xj = pl.kernel(
        functools.partial(_gather_body, epw, nchunk),
        out_type=jax.ShapeDtypeStruct((eE, vin), jnp.float32),
        mesh=mesh,
        scratch_types=[
            pltpu.VMEM((nchunk, CHUNK), jnp.int32),
            pltpu.VMEM((CHUNK, vin), jnp.float32),
            pltpu.SemaphoreType.DMA,
        ],
        compiler_params=pltpu.CompilerParams(use_tc_tiling_on_sc=False),
    )(x, src3)

    be = 512
    grid = eE // be
    msg_aug = pl.pallas_call(
        functools.partial(_edge_body, vin, vout, be),
        grid=(grid,),
        in_specs=[
            pl.BlockSpec((be, ein), lambda j: (j, 0)),
            pl.BlockSpec((be, vin), lambda j: (j, 0)),
            pl.BlockSpec((h, ein), lambda j: (0, 0)),
            pl.BlockSpec((h, 1), lambda j: (0, 0)),
            pl.BlockSpec((vin * vout, h), lambda j: (0, 0)),
            pl.BlockSpec((vin * vout, 1), lambda j: (0, 0)),
        ],
        out_specs=pl.BlockSpec((be, AUG), lambda j: (j, 0)),
        out_shape=jax.ShapeDtypeStruct((eE, AUG), jnp.float32),
    )(e, xj, W1.T, b1.reshape(h, 1), W2.T.astype(jnp.bfloat16),
      b2.reshape(vin * vout, 1))

    zeros = jnp.zeros((n, AUG), jnp.float32)
    parts = pl.kernel(
        functools.partial(_scatter_body, epw, nchunk),
        out_type=jax.ShapeDtypeStruct((NC, n, AUG), jnp.float32),
        mesh=mesh,
        scratch_types=[
            pltpu.VMEM_SHARED((n, AUG), jnp.float32),
            pltpu.VMEM((nchunk, CHUNK), jnp.int32),
            pltpu.VMEM((CHUNK, AUG), jnp.float32),
        ],
        compiler_params=pltpu.CompilerParams(use_tc_tiling_on_sc=False),
    )(msg_aug, dst3, zeros)

    bn = 1000
    out = pl.pallas_call(
        functools.partial(_final_body, vout),
        grid=(n // bn,),
        in_specs=[
            pl.BlockSpec((NC, bn, AUG), lambda j: (0, j, 0)),
            pl.BlockSpec((bn, vin), lambda j: (j, 0)),
            pl.BlockSpec((vin, vout), lambda j: (0, 0)),
            pl.BlockSpec((1, vout), lambda j: (0, 0)),
        ],
        out_specs=pl.BlockSpec((bn, vout), lambda j: (j, 0)),
        out_shape=jax.ShapeDtypeStruct((n, vout), jnp.float32),
    )(parts, x, root, bias.reshape(1, vout))

    return out


# speedup vs baseline: 1.2953x; 1.2953x over previous
import functools

import jax
import jax.numpy as jnp
from jax import lax
from jax.experimental import pallas as pl
from jax.experimental.pallas import tpu as pltpu
from jax.experimental.pallas import tpu_sc as plsc

NC = 2
NS = 16
NW = NC * NS

CHUNK = 80
CW = 8


def _gather_body(epw, nchunk, x_hbm, src_hbm, xj_hbm, idx_v, rows_v, sem):
    c = lax.axis_index("c")
    s = lax.axis_index("s")
    wid = s * NC + c
    pltpu.sync_copy(src_hbm.at[wid], idx_v)

    def step(j, carry):
        pltpu.async_copy(x_hbm.at[idx_v.at[j]], rows_v, sem).wait()
        pltpu.sync_copy(rows_v, xj_hbm.at[pl.ds(wid * epw + j * CHUNK, CHUNK)])
        return carry

    lax.fori_loop(0, nchunk, step, 0)


def _scatter_body(epw, nchunk, msg_hbm, dst_hbm, zeros_hbm, czeros_hbm,
                  ones_hbm, parts_hbm, cparts_hbm, acc_sh, cnt_sh, idx_v,
                  rows_v, ones_v):
    c = lax.axis_index("c")
    s = lax.axis_index("s")
    wid = s * NC + c

    @pl.when(s == 0)
    def _():
        pltpu.sync_copy(zeros_hbm, acc_sh)
        pltpu.sync_copy(czeros_hbm, cnt_sh)

    plsc.subcore_barrier()
    pltpu.sync_copy(dst_hbm.at[wid], idx_v)
    pltpu.sync_copy(ones_hbm, ones_v)

    def step(j, carry):
        pltpu.sync_copy(msg_hbm.at[pl.ds(wid * epw + j * CHUNK, CHUNK)], rows_v)
        pltpu.sync_copy(rows_v, acc_sh.at[idx_v.at[j]], add=True)
        pltpu.sync_copy(ones_v, cnt_sh.at[idx_v.at[j]], add=True)
        return carry

    lax.fori_loop(0, nchunk, step, 0)
    plsc.subcore_barrier()

    @pl.when(s == 0)
    def _():
        pltpu.sync_copy(acc_sh, parts_hbm.at[c])
        pltpu.sync_copy(cnt_sh, cparts_hbm.at[c])


def _edge_body(vin, vout, be, eT_ref, xj_ref, W1T_ref, b1_ref, W2T_ref,
               b2_ref, out_ref):
    et = eT_ref[...]
    hT = jnp.maximum(
        jnp.dot(W1T_ref[...], et, preferred_element_type=jnp.float32)
        + b1_ref[...], 0.0)
    WeT = jnp.dot(W2T_ref[...], hT.astype(jnp.bfloat16),
                  preferred_element_type=jnp.float32) + b2_ref[...]
    buf = xj_ref[...]
    xjT = jnp.concatenate(
        [buf[:, k * vin:(k + 1) * vin].T for k in range(4)], axis=1)
    acc = jnp.zeros((vout, be), jnp.float32)
    for i in range(vin):
        acc = acc + WeT[i * vout:(i + 1) * vout, :] * xjT[i:i + 1, :]
    out_ref[...] = jnp.concatenate(
        [acc[:, k * 128:(k + 1) * 128].T for k in range(4)], axis=1)


def _final_body(vout, parts_ref, cnt_ref, x_ref, root_ref, bias_ref, out_ref):
    s = parts_ref[0] + parts_ref[1]
    cnt = cnt_ref[0, :, 0:1] + cnt_ref[1, :, 0:1]
    mean = s / jnp.maximum(cnt, 1.0)
    y = mean + jnp.dot(x_ref[...], root_ref[...],
                       preferred_element_type=jnp.float32) + bias_ref[...]
    out_ref[...] = jnp.where(y > 0.0, y, jnp.exp(y) - 1.0)


def kernel(x, e_idx, e, W1, b1, W2, b2, root, bias):
    n, vin = x.shape
    eE, ein = e.shape
    h = W1.shape[1]
    vout = W2.shape[1] // vin

    epw = eE // NW
    nchunk = epw // CHUNK
    assert epw * NW == eE and nchunk * CHUNK == epw

    src3 = e_idx[0].reshape(NW, nchunk, CHUNK)
    dst3 = e_idx[1].reshape(NW, nchunk, CHUNK)

    mesh = plsc.VectorSubcoreMesh(core_axis_name="c", subcore_axis_name="s",
                                  num_cores=NC, num_subcores=NS)

    xj = pl.kernel(
        functools.partial(_gather_body, epw, nchunk),
        out_type=jax.ShapeDtypeStruct((eE, vin), jnp.float32),
        mesh=mesh,
        scratch_types=[
            pltpu.VMEM((nchunk, CHUNK), jnp.int32),
            pltpu.VMEM((CHUNK, vin), jnp.float32),
            pltpu.SemaphoreType.DMA,
        ],
        compiler_params=pltpu.CompilerParams(use_tc_tiling_on_sc=False),
    )(x, src3)
    xj128 = xj.reshape(eE // 4, 128)

    be = 512
    grid = eE // be
    eP = e.T.reshape(ein, grid, 128, 4).transpose(0, 1, 3, 2).reshape(ein, eE)
    msg = pl.pallas_call(
        functools.partial(_edge_body, vin, vout, be),
        grid=(grid,),
        in_specs=[
            pl.BlockSpec((ein, be), lambda j: (0, j)),
            pl.BlockSpec((be // 4, 128), lambda j: (j, 0)),
            pl.BlockSpec((h, ein), lambda j: (0, 0)),
            pl.BlockSpec((h, 1), lambda j: (0, 0)),
            pl.BlockSpec((vin * vout, h), lambda j: (0, 0)),
            pl.BlockSpec((vin * vout, 1), lambda j: (0, 0)),
        ],
        out_specs=pl.BlockSpec((be // 4, 128), lambda j: (j, 0)),
        out_shape=jax.ShapeDtypeStruct((eE // 4, 128), jnp.float32),
    )(eP, xj128, W1.T, b1.reshape(h, 1), W2.T.astype(jnp.bfloat16),
      b2.reshape(vin * vout, 1))
    msg32 = msg.reshape(eE, vout)

    zeros = jnp.zeros((n, vout), jnp.float32)
    czeros = jnp.zeros((n, CW), jnp.float32)
    ones = jnp.ones((CHUNK, CW), jnp.float32)
    parts, cparts = pl.kernel(
        functools.partial(_scatter_body, epw, nchunk),
        out_type=(jax.ShapeDtypeStruct((NC, n, vout), jnp.float32),
                  jax.ShapeDtypeStruct((NC, n, CW), jnp.float32)),
        mesh=mesh,
        scratch_types=[
            pltpu.VMEM_SHARED((n, vout), jnp.float32),
            pltpu.VMEM_SHARED((n, CW), jnp.float32),
            pltpu.VMEM((nchunk, CHUNK), jnp.int32),
            pltpu.VMEM((CHUNK, vout), jnp.float32),
            pltpu.VMEM((CHUNK, CW), jnp.float32),
        ],
        compiler_params=pltpu.CompilerParams(use_tc_tiling_on_sc=False),
    )(msg32, dst3, zeros, czeros, ones)

    bn = 1000
    out = pl.pallas_call(
        functools.partial(_final_body, vout),
        grid=(n // bn,),
        in_specs=[
            pl.BlockSpec((NC, bn, vout), lambda j: (0, j, 0)),
            pl.BlockSpec((NC, bn, CW), lambda j: (0, j, 0)),
            pl.BlockSpec((bn, vin), lambda j: (j, 0)),
            pl.BlockSpec((vin, vout), lambda j: (0, 0)),
            pl.BlockSpec((1, vout), lambda j: (0, 0)),
        ],
        out_specs=pl.BlockSpec((bn, vout), lambda j: (j, 0)),
        out_shape=jax.ShapeDtypeStruct((n, vout), jnp.float32),
    )(parts, cparts, x, root, bias.reshape(1, vout))

    return out


# speedup vs baseline: 1.9318x; 1.4914x over previous
import functools

import jax
import jax.numpy as jnp
from jax import lax
from jax.experimental import pallas as pl
from jax.experimental.pallas import tpu as pltpu
from jax.experimental.pallas import tpu_sc as plsc

NC = 2
NS = 16
NW = NC * NS

CHUNK = 80
CW = 8


def _gather_body(epw, nchunk, x_hbm, src_hbm, xj_hbm, idx_v, rows_v, sem):
    c = lax.axis_index("c")
    s = lax.axis_index("s")
    wid = s * NC + c
    pltpu.sync_copy(src_hbm.at[wid], idx_v)

    def step(j, carry):
        pltpu.async_copy(x_hbm.at[idx_v.at[j]], rows_v, sem).wait()
        pltpu.sync_copy(rows_v, xj_hbm.at[pl.ds(wid * epw + j * CHUNK, CHUNK)])
        return carry

    lax.fori_loop(0, nchunk, step, 0)


def _scatter_body(epw, nchunk, msg_hbm, dst_hbm, zeros_hbm, czeros_hbm,
                  ones_hbm, parts_hbm, cparts_hbm, acc_sh, cnt_sh, idx_v,
                  rows_v, ones_v):
    c = lax.axis_index("c")
    s = lax.axis_index("s")
    wid = s * NC + c

    @pl.when(s == 0)
    def _():
        pltpu.sync_copy(zeros_hbm, acc_sh)
        pltpu.sync_copy(czeros_hbm, cnt_sh)

    plsc.subcore_barrier()
    pltpu.sync_copy(dst_hbm.at[wid], idx_v)
    pltpu.sync_copy(ones_hbm, ones_v)

    def step(j, carry):
        pltpu.sync_copy(msg_hbm.at[pl.ds(wid * epw + j * CHUNK, CHUNK)], rows_v)
        pltpu.sync_copy(rows_v, acc_sh.at[idx_v.at[j]], add=True)
        pltpu.sync_copy(ones_v, cnt_sh.at[idx_v.at[j]], add=True)
        return carry

    lax.fori_loop(0, nchunk, step, 0)
    plsc.subcore_barrier()

    @pl.when(s == 0)
    def _():
        pltpu.sync_copy(acc_sh, parts_hbm.at[c])
        pltpu.sync_copy(cnt_sh, cparts_hbm.at[c])


def _edge_body(vin, vout, be, eT_ref, xj_ref, W1T_ref, b1_ref, W2T_ref,
               b2_ref, out_ref):
    et = eT_ref[...]
    hT = jnp.maximum(
        jnp.dot(W1T_ref[...], et, preferred_element_type=jnp.float32)
        + b1_ref[...], 0.0)
    WeT = jnp.dot(W2T_ref[...], hT.astype(jnp.bfloat16),
                  preferred_element_type=jnp.float32) + b2_ref[...]
    buf = xj_ref[...]
    xjT = jnp.concatenate(
        [buf[:, k * vin:(k + 1) * vin].T for k in range(4)], axis=1)
    acc = jnp.zeros((vout, be), jnp.float32)
    for i in range(vin):
        acc = acc + WeT[i * vout:(i + 1) * vout, :] * xjT[i:i + 1, :]
    q = be // 4
    out_ref[...] = jnp.concatenate(
        [acc[:, k * q:(k + 1) * q].T for k in range(4)], axis=1)


def _final_body(vout, parts_ref, cnt_ref, x_ref, root_ref, bias_ref, out_ref):
    s = parts_ref[0] + parts_ref[1]
    cnt = cnt_ref[0, :, 0:1] + cnt_ref[1, :, 0:1]
    mean = s / jnp.maximum(cnt, 1.0)
    y = mean + jnp.dot(x_ref[...], root_ref[...],
                       preferred_element_type=jnp.float32) + bias_ref[...]
    out_ref[...] = jnp.where(y > 0.0, y, jnp.exp(y) - 1.0)


def kernel(x, e_idx, e, W1, b1, W2, b2, root, bias):
    n, vin = x.shape
    eE, ein = e.shape
    h = W1.shape[1]
    vout = W2.shape[1] // vin

    epw = eE // NW
    nchunk = epw // CHUNK
    assert epw * NW == eE and nchunk * CHUNK == epw

    src3 = e_idx[0].reshape(NW, nchunk, CHUNK)
    dst3 = e_idx[1].reshape(NW, nchunk, CHUNK)

    mesh = plsc.VectorSubcoreMesh(core_axis_name="c", subcore_axis_name="s",
                                  num_cores=NC, num_subcores=NS)

    xj = pl.kernel(
        functools.partial(_gather_body, epw, nchunk),
        out_type=jax.ShapeDtypeStruct((eE, vin), jnp.float32),
        mesh=mesh,
        scratch_types=[
            pltpu.VMEM((nchunk, CHUNK), jnp.int32),
            pltpu.VMEM((CHUNK, vin), jnp.float32),
            pltpu.SemaphoreType.DMA,
        ],
        compiler_params=pltpu.CompilerParams(use_tc_tiling_on_sc=False),
    )(x, src3)
    xj128 = xj.reshape(eE // 4, 128)

    be = 2560
    grid = eE // be
    eP = e.T.reshape(ein, grid, be // 4, 4).transpose(0, 1, 3, 2) \
        .reshape(ein, eE)
    msg = pl.pallas_call(
        functools.partial(_edge_body, vin, vout, be),
        grid=(grid,),
        in_specs=[
            pl.BlockSpec((ein, be), lambda j: (0, j)),
            pl.BlockSpec((be // 4, 128), lambda j: (j, 0)),
            pl.BlockSpec((h, ein), lambda j: (0, 0)),
            pl.BlockSpec((h, 1), lambda j: (0, 0)),
            pl.BlockSpec((vin * vout, h), lambda j: (0, 0)),
            pl.BlockSpec((vin * vout, 1), lambda j: (0, 0)),
        ],
        out_specs=pl.BlockSpec((be // 4, 128), lambda j: (j, 0)),
        out_shape=jax.ShapeDtypeStruct((eE // 4, 128), jnp.float32),
    )(eP, xj128, W1.T, b1.reshape(h, 1), W2.T.astype(jnp.bfloat16),
      b2.reshape(vin * vout, 1))
    msg32 = msg.reshape(eE, vout)

    zeros = jnp.zeros((n, vout), jnp.float32)
    czeros = jnp.zeros((n, CW), jnp.float32)
    ones = jnp.ones((CHUNK, CW), jnp.float32)
    parts, cparts = pl.kernel(
        functools.partial(_scatter_body, epw, nchunk),
        out_type=(jax.ShapeDtypeStruct((NC, n, vout), jnp.float32),
                  jax.ShapeDtypeStruct((NC, n, CW), jnp.float32)),
        mesh=mesh,
        scratch_types=[
            pltpu.VMEM_SHARED((n, vout), jnp.float32),
            pltpu.VMEM_SHARED((n, CW), jnp.float32),
            pltpu.VMEM((nchunk, CHUNK), jnp.int32),
            pltpu.VMEM((CHUNK, vout), jnp.float32),
            pltpu.VMEM((CHUNK, CW), jnp.float32),
        ],
        compiler_params=pltpu.CompilerParams(use_tc_tiling_on_sc=False),
    )(msg32, dst3, zeros, czeros, ones)

    bn = 1000
    out = pl.pallas_call(
        functools.partial(_final_body, vout),
        grid=(n // bn,),
        in_specs=[
            pl.BlockSpec((NC, bn, vout), lambda j: (0, j, 0)),
            pl.BlockSpec((NC, bn, CW), lambda j: (0, j, 0)),
            pl.BlockSpec((bn, vin), lambda j: (j, 0)),
            pl.BlockSpec((vin, vout), lambda j: (0, 0)),
            pl.BlockSpec((1, vout), lambda j: (0, 0)),
        ],
        out_specs=pl.BlockSpec((bn, vout), lambda j: (j, 0)),
        out_shape=jax.ShapeDtypeStruct((n, vout), jnp.float32),
    )(parts, cparts, x, root, bias.reshape(1, vout))

    return out
